# baseline (speedup 1.0000x reference)
import functools

import jax
import jax.numpy as jnp
from jax import lax
from jax.experimental import pallas as pl
from jax.experimental.pallas import tpu as pltpu
from jax.experimental.pallas import tpu_sc as plsc

TOKENS = 8192
INPUT_DIM = 4096
HIDDEN_DIM = 256
N_EXPERTS = 64

BLOCK = 1024

NCHUNK = 4
CTOK = TOKENS // NCHUNK
NWORKERS = 32
TOK_PER_W = CTOK // NWORKERS
NGRP = TOK_PER_W // 16
LWORDS = TOK_PER_W * N_EXPERTS


def _logits_kernel(x_ref, w1_ref, b1_ref, w2_ref, b2_ref, logits_ref):
    h = jax.lax.dot_general(
        x_ref[...], w1_ref[...], (((1,), (1,)), ((), ())),
        preferred_element_type=jnp.float32)
    h = jnp.maximum(h + b1_ref[...], 0.0)
    logits = jax.lax.dot_general(
        h, w2_ref[...], (((1,), (1,)), ((), ())),
        preferred_element_type=jnp.float32)
    logits_ref[...] = logits + b2_ref[...]


def _tc_logits(x, W1, b1, W2, b2, chunk):
    off = chunk * (CTOK // BLOCK)
    return pl.pallas_call(
        _logits_kernel,
        grid=(CTOK // BLOCK,),
        in_specs=[
            pl.BlockSpec((BLOCK, INPUT_DIM), lambda i: (i + off, 0)),
            pl.BlockSpec((HIDDEN_DIM, INPUT_DIM), lambda i: (0, 0)),
            pl.BlockSpec((1, HIDDEN_DIM), lambda i: (0, 0)),
            pl.BlockSpec((N_EXPERTS, HIDDEN_DIM), lambda i: (0, 0)),
            pl.BlockSpec((1, N_EXPERTS), lambda i: (0, 0)),
        ],
        out_specs=pl.BlockSpec((BLOCK, N_EXPERTS), lambda i: (i, 0)),
        out_shape=jax.ShapeDtypeStruct((CTOK, N_EXPERTS), jnp.float32),
    )(x, W1, b1.reshape(1, HIDDEN_DIM), W2, b2.reshape(1, N_EXPERTS))


def _route_body(logits_hbm, zeros_hbm, gates_hbm, idx_hbm, lbuf, gbuf, ibuf):
    wid = lax.axis_index("s") * 2 + lax.axis_index("c")
    base = wid * LWORDS
    pltpu.sync_copy(logits_hbm.at[pl.ds(base, LWORDS)], lbuf)
    pltpu.sync_copy(zeros_hbm, gbuf)

    iota = lax.iota(jnp.int32, 16)
    neg_inf = jnp.full((16,), -jnp.inf, jnp.float32)
    zero_i = jnp.zeros((16,), jnp.int32)
    one_i = jnp.full((16,), 1, jnp.int32)
    one_f = jnp.full((16,), 1.0, jnp.float32)

    def group(g, carry):
        rows = g * 16 + iota
        rows64 = rows * N_EXPERTS
        m1 = neg_inf
        m2 = neg_inf
        i1 = zero_i
        i2 = zero_i
        for e in range(N_EXPERTS):
            e_vec = jnp.full((16,), e, jnp.int32)
            v = plsc.load_gather(lbuf, [rows64 + e_vec])
            gt1 = v > m1
            gt2 = jnp.logical_and(v > m2, jnp.logical_not(gt1))
            i2 = jnp.where(gt1, i1, jnp.where(gt2, e_vec, i2))
            m2 = jnp.where(gt1, m1, jnp.where(gt2, v, m2))
            i1 = jnp.where(gt1, e_vec, i1)
            m1 = jnp.where(gt1, v, m1)
        e2 = jnp.exp(m2 - m1)
        den = one_f + e2
        g1 = one_f / den
        g2 = e2 / den
        plsc.store_scatter(gbuf, [rows64 + i1], g1)
        plsc.store_scatter(gbuf, [rows64 + i2], g2)
        rows2 = rows * 2
        plsc.store_scatter(ibuf, [rows2], i1)
        plsc.store_scatter(ibuf, [rows2 + one_i], i2)
        return carry

    lax.fori_loop(0, NGRP, group, 0)

    pltpu.sync_copy(gbuf, gates_hbm.at[pl.ds(base, LWORDS)])
    pltpu.sync_copy(ibuf, idx_hbm.at[pl.ds(wid * TOK_PER_W * 2, TOK_PER_W * 2)])


def _sc_route(logits_flat, zeros_w):
    mesh = plsc.VectorSubcoreMesh(core_axis_name="c", subcore_axis_name="s")
    run = pl.kernel(
        _route_body,
        mesh=mesh,
        out_type=[
            jax.ShapeDtypeStruct((CTOK * N_EXPERTS,), jnp.float32),
            jax.ShapeDtypeStruct((CTOK * 2,), jnp.int32),
        ],
        scratch_types=[
            pltpu.VMEM((LWORDS,), jnp.float32),
            pltpu.VMEM((LWORDS,), jnp.float32),
            pltpu.VMEM((TOK_PER_W * 2,), jnp.int32),
        ],
        compiler_params=pltpu.CompilerParams(needs_layout_passes=False),
    )
    return run(logits_flat, zeros_w)


@jax.jit
def kernel(x, W1, b1, W2, b2):
    zeros_w = jnp.zeros((LWORDS,), jnp.float32)
    gates_parts = []
    idx_parts = []
    for c in range(NCHUNK):
        logits_c = _tc_logits(x, W1, b1, W2, b2, c)
        g_c, i_c = _sc_route(logits_c.reshape(-1), zeros_w)
        gates_parts.append(g_c.reshape(CTOK, N_EXPERTS))
        idx_parts.append(i_c.reshape(CTOK, 2))
    return (jnp.concatenate(gates_parts, axis=0),
            jnp.concatenate(idx_parts, axis=0))


# speedup vs baseline: 2.0759x; 2.0759x over previous
import functools

import jax
import jax.numpy as jnp
from jax.experimental import pallas as pl

TOKENS = 8192
INPUT_DIM = 4096
HIDDEN_DIM = 256
N_EXPERTS = 64

BLOCK = 1024


def _gate_kernel(x_ref, w1_ref, b1_ref, w2_ref, b2_ref, gates_ref, idx_ref):
    x = x_ref[...]
    h = jax.lax.dot_general(
        x, w1_ref[...], (((1,), (1,)), ((), ())),
        preferred_element_type=jnp.float32)
    h = jnp.maximum(h + b1_ref[...], 0.0)
    logits = jax.lax.dot_general(
        h, w2_ref[...], (((1,), (1,)), ((), ())),
        preferred_element_type=jnp.float32)
    logits = logits + b2_ref[...]

    lanes = jax.lax.broadcasted_iota(jnp.int32, logits.shape, 1)
    l1 = jnp.max(logits, axis=-1, keepdims=True)
    i1 = jnp.argmax(logits, axis=-1).astype(jnp.int32)
    masked = jnp.where(lanes == i1[:, None], -jnp.inf, logits)
    l2 = jnp.max(masked, axis=-1, keepdims=True)
    i2 = jnp.argmax(masked, axis=-1).astype(jnp.int32)

    e = jnp.exp(l2 - l1)
    denom = 1.0 + e
    g1 = 1.0 / denom
    g2 = e / denom

    gates = jnp.where(lanes == i1[:, None], g1, 0.0)
    gates = jnp.where(lanes == i2[:, None], g2, gates)
    gates_ref[...] = gates
    idx_ref[...] = jnp.stack([i1, i2], axis=-1)


@jax.jit
def kernel(x, W1, b1, W2, b2):
    grid = (TOKENS // BLOCK,)
    gates, idx = pl.pallas_call(
        _gate_kernel,
        grid=grid,
        in_specs=[
            pl.BlockSpec((BLOCK, INPUT_DIM), lambda i: (i, 0)),
            pl.BlockSpec((HIDDEN_DIM, INPUT_DIM), lambda i: (0, 0)),
            pl.BlockSpec((1, HIDDEN_DIM), lambda i: (0, 0)),
            pl.BlockSpec((N_EXPERTS, HIDDEN_DIM), lambda i: (0, 0)),
            pl.BlockSpec((1, N_EXPERTS), lambda i: (0, 0)),
        ],
        out_specs=[
            pl.BlockSpec((BLOCK, N_EXPERTS), lambda i: (i, 0)),
            pl.BlockSpec((BLOCK, 2), lambda i: (i, 0)),
        ],
        out_shape=[
            jax.ShapeDtypeStruct((TOKENS, N_EXPERTS), jnp.float32),
            jax.ShapeDtypeStruct((TOKENS, 2), jnp.int32),
        ],
    )(x, W1, b1.reshape(1, HIDDEN_DIM), W2, b2.reshape(1, N_EXPERTS))
    return gates, idx
